# Initial kernel scaffold; baseline (speedup 1.0000x reference)
import jax
import jax.numpy as jnp
from jax.experimental import pallas as pl


def kernel(x, edge_attr, w_node, b_node, w_edge, b_edge, bn_gamma, bn_beta):
    raise NotImplementedError("write your pallas kernel here")


# speedup vs baseline: 1.3118x; 1.3118x over previous
import jax
import jax.numpy as jnp
from jax import lax
from jax.experimental import pallas as pl
from jax.experimental.pallas import tpu as pltpu

_TILE_ROWS = 1024
_VMEM_LIMIT_BYTES = 32 * 1024 * 1024


def _cdiv(a, b):
    return -(-a // b)


def _pad_rows(a, rows_padded):
    r = a.shape[0]
    if rows_padded != r:
        a = jnp.pad(a, ((0, rows_padded - r), (0, 0)))
    return a


def _encode_stats_kernel(x_ref, wn_ref, bn_ref, e_ref, we_ref,
                         xo_ref, s1_ref, s2_ref):
    @pl.when(pl.program_id(0) == 0)
    def _init():
        s1_ref[...] = jnp.zeros_like(s1_ref)
        s2_ref[...] = jnp.zeros_like(s2_ref)

    xo_ref[...] = (jnp.dot(x_ref[...], wn_ref[...],
                           preferred_element_type=jnp.float32)
                   + bn_ref[...]).astype(xo_ref.dtype)
    acc = jnp.dot(e_ref[...], we_ref[...], preferred_element_type=jnp.float32)
    s1_ref[...] += jnp.sum(acc, axis=0, keepdims=True)
    s2_ref[...] += jnp.sum(acc * acc, axis=0, keepdims=True)


def _edge_norm_kernel(e_ref, we_ref, ss_ref, o_ref):
    acc = jnp.dot(e_ref[...], we_ref[...], preferred_element_type=jnp.float32)
    o_ref[...] = (acc * ss_ref[0:1, :] + ss_ref[1:2, :]).astype(o_ref.dtype)


def kernel(x, edge_attr, w_node, b_node, w_edge, b_edge, bn_gamma, bn_beta):
    eps = 1e-5
    n, _ = x.shape
    r_e, _ = edge_attr.shape
    dout = w_node.shape[1]

    tiles = max(_cdiv(max(n, r_e), _TILE_ROWS), 1)
    rp = tiles * _TILE_ROWS

    xf = _pad_rows(x.astype(jnp.float32), rp)
    ef = _pad_rows(edge_attr.astype(jnp.float32), rp)
    wn = w_node.astype(jnp.float32)
    we = w_edge.astype(jnp.float32)
    bn = b_node.astype(jnp.float32).reshape(1, dout)

    x_enc, s1, s2 = pl.pallas_call(
        _encode_stats_kernel,
        out_shape=(jax.ShapeDtypeStruct((rp, dout), jnp.float32),
                   jax.ShapeDtypeStruct((1, dout), jnp.float32),
                   jax.ShapeDtypeStruct((1, dout), jnp.float32)),
        grid_spec=pltpu.PrefetchScalarGridSpec(
            num_scalar_prefetch=0,
            grid=(tiles,),
            in_specs=[
                pl.BlockSpec((_TILE_ROWS, dout), lambda i: (i, 0)),
                pl.BlockSpec(wn.shape, lambda i: (0, 0)),
                pl.BlockSpec((1, dout), lambda i: (0, 0)),
                pl.BlockSpec((_TILE_ROWS, dout), lambda i: (i, 0)),
                pl.BlockSpec(we.shape, lambda i: (0, 0)),
            ],
            out_specs=[
                pl.BlockSpec((_TILE_ROWS, dout), lambda i: (i, 0)),
                pl.BlockSpec((1, dout), lambda i: (0, 0)),
                pl.BlockSpec((1, dout), lambda i: (0, 0)),
            ],
        ),
        compiler_params=pltpu.CompilerParams(
            dimension_semantics=("arbitrary",),
            vmem_limit_bytes=_VMEM_LIMIT_BYTES),
    )(xf, wn, bn, ef, we)
    x_enc = x_enc[:n] if rp != n else x_enc

    cnt = jnp.float32(max(r_e, 1))
    mu = s1.reshape(dout) / cnt
    var = jnp.maximum(s2.reshape(dout) / cnt - mu * mu, 0.0)
    scale = bn_gamma.astype(jnp.float32) * lax.rsqrt(var + eps)
    shift = bn_beta.astype(jnp.float32) - mu * scale
    ss = jnp.stack([scale, shift])

    e_enc = pl.pallas_call(
        _edge_norm_kernel,
        out_shape=jax.ShapeDtypeStruct((rp, dout), jnp.float32),
        grid_spec=pltpu.PrefetchScalarGridSpec(
            num_scalar_prefetch=0,
            grid=(tiles,),
            in_specs=[
                pl.BlockSpec((_TILE_ROWS, dout), lambda i: (i, 0)),
                pl.BlockSpec(we.shape, lambda i: (0, 0)),
                pl.BlockSpec((2, dout), lambda i: (0, 0)),
            ],
            out_specs=pl.BlockSpec((_TILE_ROWS, dout), lambda i: (i, 0)),
        ),
        compiler_params=pltpu.CompilerParams(
            dimension_semantics=("parallel",),
            vmem_limit_bytes=_VMEM_LIMIT_BYTES),
    )(ef, we, ss)
    e_enc = e_enc[:r_e] if rp != r_e else e_enc

    return {"x": x_enc, "edge_attr": e_enc}


# speedup vs baseline: 1.8800x; 1.4332x over previous
import jax
import jax.numpy as jnp
from jax import lax
from jax.experimental import pallas as pl
from jax.experimental.pallas import tpu as pltpu

_TILE_ROWS = 2048
_VMEM_LIMIT_BYTES = 32 * 1024 * 1024


def _cdiv(a, b):
    return -(-a // b)


def _pad_rows(a, rows_padded):
    r = a.shape[0]
    if rows_padded != r:
        a = jnp.pad(a, ((0, rows_padded - r), (0, 0)))
    return a


def _encode_stats_kernel(x_ref, wn_ref, bn_ref, e_ref, we_ref,
                         xo_ref, s1_ref, s2_ref):
    @pl.when(pl.program_id(0) == 0)
    def _init():
        s1_ref[...] = jnp.zeros_like(s1_ref)
        s2_ref[...] = jnp.zeros_like(s2_ref)

    xo_ref[...] = (jnp.dot(x_ref[...], wn_ref[...],
                           preferred_element_type=jnp.float32)
                   + bn_ref[...]).astype(xo_ref.dtype)
    acc = jnp.dot(e_ref[...], we_ref[...], preferred_element_type=jnp.float32)
    s1_ref[...] += jnp.sum(acc, axis=0, keepdims=True)
    s2_ref[...] += jnp.sum(acc * acc, axis=0, keepdims=True)


def _make_edge_norm_kernel(inv_cnt, eps):
    def _edge_norm_kernel(e_ref, we_ref, s1_ref, s2_ref, g_ref, b_ref, o_ref):
        mu = s1_ref[...] * inv_cnt
        var = jnp.maximum(s2_ref[...] * inv_cnt - mu * mu, 0.0)
        scale = g_ref[...] * lax.rsqrt(var + eps)
        shift = b_ref[...] - mu * scale
        acc = jnp.dot(e_ref[...], we_ref[...],
                      preferred_element_type=jnp.float32)
        o_ref[...] = (acc * scale + shift).astype(o_ref.dtype)
    return _edge_norm_kernel


def kernel(x, edge_attr, w_node, b_node, w_edge, b_edge, bn_gamma, bn_beta):
    eps = 1e-5
    n, _ = x.shape
    r_e, _ = edge_attr.shape
    dout = w_node.shape[1]

    tiles = max(_cdiv(max(n, r_e), _TILE_ROWS), 1)
    rp = tiles * _TILE_ROWS

    xf = _pad_rows(x.astype(jnp.float32), rp)
    ef = _pad_rows(edge_attr.astype(jnp.float32), rp)
    wn = w_node.astype(jnp.float32)
    we = w_edge.astype(jnp.float32)
    bn = b_node.astype(jnp.float32).reshape(1, dout)
    row = lambda v: v.astype(jnp.float32).reshape(1, dout)

    x_enc, s1, s2 = pl.pallas_call(
        _encode_stats_kernel,
        out_shape=(jax.ShapeDtypeStruct((rp, dout), jnp.float32),
                   jax.ShapeDtypeStruct((1, dout), jnp.float32),
                   jax.ShapeDtypeStruct((1, dout), jnp.float32)),
        grid_spec=pltpu.PrefetchScalarGridSpec(
            num_scalar_prefetch=0,
            grid=(tiles,),
            in_specs=[
                pl.BlockSpec((_TILE_ROWS, dout), lambda i: (i, 0)),
                pl.BlockSpec(wn.shape, lambda i: (0, 0)),
                pl.BlockSpec((1, dout), lambda i: (0, 0)),
                pl.BlockSpec((_TILE_ROWS, dout), lambda i: (i, 0)),
                pl.BlockSpec(we.shape, lambda i: (0, 0)),
            ],
            out_specs=[
                pl.BlockSpec((_TILE_ROWS, dout), lambda i: (i, 0)),
                pl.BlockSpec((1, dout), lambda i: (0, 0)),
                pl.BlockSpec((1, dout), lambda i: (0, 0)),
            ],
        ),
        compiler_params=pltpu.CompilerParams(
            dimension_semantics=("arbitrary",),
            vmem_limit_bytes=_VMEM_LIMIT_BYTES),
    )(xf, wn, bn, ef, we)
    x_enc = x_enc[:n] if rp != n else x_enc

    inv_cnt = 1.0 / float(max(r_e, 1))
    e_enc = pl.pallas_call(
        _make_edge_norm_kernel(inv_cnt, eps),
        out_shape=jax.ShapeDtypeStruct((rp, dout), jnp.float32),
        grid_spec=pltpu.PrefetchScalarGridSpec(
            num_scalar_prefetch=0,
            grid=(tiles,),
            in_specs=[
                pl.BlockSpec((_TILE_ROWS, dout), lambda i: (i, 0)),
                pl.BlockSpec(we.shape, lambda i: (0, 0)),
                pl.BlockSpec((1, dout), lambda i: (0, 0)),
                pl.BlockSpec((1, dout), lambda i: (0, 0)),
                pl.BlockSpec((1, dout), lambda i: (0, 0)),
                pl.BlockSpec((1, dout), lambda i: (0, 0)),
            ],
            out_specs=pl.BlockSpec((_TILE_ROWS, dout), lambda i: (i, 0)),
        ),
        compiler_params=pltpu.CompilerParams(
            dimension_semantics=("parallel",),
            vmem_limit_bytes=_VMEM_LIMIT_BYTES),
    )(ef, we, s1, s2, row(bn_gamma), row(bn_beta))
    e_enc = e_enc[:r_e] if rp != r_e else e_enc

    return {"x": x_enc, "edge_attr": e_enc}


# speedup vs baseline: 2.5244x; 1.3428x over previous
import jax
import jax.numpy as jnp
from jax import lax
from jax.experimental import pallas as pl
from jax.experimental.pallas import tpu as pltpu

_TILE_ROWS = 4096
_VMEM_LIMIT_BYTES = 32 * 1024 * 1024


def _cdiv(a, b):
    return -(-a // b)


def _pad_rows(a, rows_padded):
    r = a.shape[0]
    if rows_padded != r:
        a = jnp.pad(a, ((0, rows_padded - r), (0, 0)))
    return a


def _encode_stats_kernel(x_ref, wn_ref, bn_ref, e_ref, we_ref,
                         xo_ref, s1_ref, s2_ref):
    @pl.when(pl.program_id(0) == 0)
    def _init():
        s1_ref[...] = jnp.zeros_like(s1_ref)
        s2_ref[...] = jnp.zeros_like(s2_ref)

    xo_ref[...] = (jnp.dot(x_ref[...], wn_ref[...],
                           preferred_element_type=jnp.float32)
                   + bn_ref[...]).astype(xo_ref.dtype)
    acc = jnp.dot(e_ref[...], we_ref[...], preferred_element_type=jnp.float32)
    s1_ref[...] += jnp.sum(acc, axis=0, keepdims=True)
    s2_ref[...] += jnp.sum(acc * acc, axis=0, keepdims=True)


def _make_edge_norm_kernel(inv_cnt, eps):
    def _edge_norm_kernel(e_ref, we_ref, s1_ref, s2_ref, g_ref, b_ref, o_ref):
        mu = s1_ref[...] * inv_cnt
        var = jnp.maximum(s2_ref[...] * inv_cnt - mu * mu, 0.0)
        scale = g_ref[...] * lax.rsqrt(var + eps)
        shift = b_ref[...] - mu * scale
        acc = jnp.dot(e_ref[...], we_ref[...],
                      preferred_element_type=jnp.float32)
        o_ref[...] = (acc * scale + shift).astype(o_ref.dtype)
    return _edge_norm_kernel


def kernel(x, edge_attr, w_node, b_node, w_edge, b_edge, bn_gamma, bn_beta):
    eps = 1e-5
    n, _ = x.shape
    r_e, _ = edge_attr.shape
    dout = w_node.shape[1]

    tiles = max(_cdiv(max(n, r_e), _TILE_ROWS), 1)
    rp = tiles * _TILE_ROWS

    xf = _pad_rows(x.astype(jnp.float32), rp)
    ef = _pad_rows(edge_attr.astype(jnp.float32), rp)
    wn = w_node.astype(jnp.float32)
    we = w_edge.astype(jnp.float32)
    bn = b_node.astype(jnp.float32).reshape(1, dout)
    row = lambda v: v.astype(jnp.float32).reshape(1, dout)

    x_enc, s1, s2 = pl.pallas_call(
        _encode_stats_kernel,
        out_shape=(jax.ShapeDtypeStruct((rp, dout), jnp.float32),
                   jax.ShapeDtypeStruct((1, dout), jnp.float32),
                   jax.ShapeDtypeStruct((1, dout), jnp.float32)),
        grid_spec=pltpu.PrefetchScalarGridSpec(
            num_scalar_prefetch=0,
            grid=(tiles,),
            in_specs=[
                pl.BlockSpec((_TILE_ROWS, dout), lambda i: (i, 0)),
                pl.BlockSpec(wn.shape, lambda i: (0, 0)),
                pl.BlockSpec((1, dout), lambda i: (0, 0)),
                pl.BlockSpec((_TILE_ROWS, dout), lambda i: (i, 0)),
                pl.BlockSpec(we.shape, lambda i: (0, 0)),
            ],
            out_specs=[
                pl.BlockSpec((_TILE_ROWS, dout), lambda i: (i, 0)),
                pl.BlockSpec((1, dout), lambda i: (0, 0)),
                pl.BlockSpec((1, dout), lambda i: (0, 0)),
            ],
        ),
        compiler_params=pltpu.CompilerParams(
            dimension_semantics=("arbitrary",),
            vmem_limit_bytes=_VMEM_LIMIT_BYTES),
    )(xf, wn, bn, ef, we)
    x_enc = x_enc[:n] if rp != n else x_enc

    inv_cnt = 1.0 / float(max(r_e, 1))
    e_enc = pl.pallas_call(
        _make_edge_norm_kernel(inv_cnt, eps),
        out_shape=jax.ShapeDtypeStruct((rp, dout), jnp.float32),
        grid_spec=pltpu.PrefetchScalarGridSpec(
            num_scalar_prefetch=0,
            grid=(tiles,),
            in_specs=[
                pl.BlockSpec((_TILE_ROWS, dout), lambda i: (i, 0)),
                pl.BlockSpec(we.shape, lambda i: (0, 0)),
                pl.BlockSpec((1, dout), lambda i: (0, 0)),
                pl.BlockSpec((1, dout), lambda i: (0, 0)),
                pl.BlockSpec((1, dout), lambda i: (0, 0)),
                pl.BlockSpec((1, dout), lambda i: (0, 0)),
            ],
            out_specs=pl.BlockSpec((_TILE_ROWS, dout), lambda i: (i, 0)),
        ),
        compiler_params=pltpu.CompilerParams(
            dimension_semantics=("parallel",),
            vmem_limit_bytes=_VMEM_LIMIT_BYTES),
    )(ef, we, s1, s2, row(bn_gamma), row(bn_beta))
    e_enc = e_enc[:r_e] if rp != r_e else e_enc

    return {"x": x_enc, "edge_attr": e_enc}


# speedup vs baseline: 2.7716x; 1.0979x over previous
import jax
import jax.numpy as jnp
from jax import lax
from jax.experimental import pallas as pl
from jax.experimental.pallas import tpu as pltpu

_TILE_ROWS = 8192
_VMEM_LIMIT_BYTES = 32 * 1024 * 1024


def _cdiv(a, b):
    return -(-a // b)


def _pad_rows(a, rows_padded):
    r = a.shape[0]
    if rows_padded != r:
        a = jnp.pad(a, ((0, rows_padded - r), (0, 0)))
    return a


def _encode_stats_kernel(x_ref, wn_ref, bn_ref, e_ref, we_ref,
                         xo_ref, s1_ref, s2_ref):
    @pl.when(pl.program_id(0) == 0)
    def _init():
        s1_ref[...] = jnp.zeros_like(s1_ref)
        s2_ref[...] = jnp.zeros_like(s2_ref)

    xo_ref[...] = (jnp.dot(x_ref[...], wn_ref[...],
                           preferred_element_type=jnp.float32)
                   + bn_ref[...]).astype(xo_ref.dtype)
    acc = jnp.dot(e_ref[...], we_ref[...], preferred_element_type=jnp.float32)
    s1_ref[...] += jnp.sum(acc, axis=0, keepdims=True)
    s2_ref[...] += jnp.sum(acc * acc, axis=0, keepdims=True)


def _make_edge_norm_kernel(inv_cnt, eps):
    def _edge_norm_kernel(e_ref, we_ref, s1_ref, s2_ref, g_ref, b_ref, o_ref):
        mu = s1_ref[...] * inv_cnt
        var = jnp.maximum(s2_ref[...] * inv_cnt - mu * mu, 0.0)
        scale = g_ref[...] * lax.rsqrt(var + eps)
        shift = b_ref[...] - mu * scale
        acc = jnp.dot(e_ref[...], we_ref[...],
                      preferred_element_type=jnp.float32)
        o_ref[...] = (acc * scale + shift).astype(o_ref.dtype)
    return _edge_norm_kernel


def kernel(x, edge_attr, w_node, b_node, w_edge, b_edge, bn_gamma, bn_beta):
    eps = 1e-5
    n, _ = x.shape
    r_e, _ = edge_attr.shape
    dout = w_node.shape[1]

    tiles = max(_cdiv(max(n, r_e), _TILE_ROWS), 1)
    rp = tiles * _TILE_ROWS

    xf = _pad_rows(x.astype(jnp.float32), rp)
    ef = _pad_rows(edge_attr.astype(jnp.float32), rp)
    wn = w_node.astype(jnp.float32)
    we = w_edge.astype(jnp.float32)
    bn = b_node.astype(jnp.float32).reshape(1, dout)
    row = lambda v: v.astype(jnp.float32).reshape(1, dout)

    x_enc, s1, s2 = pl.pallas_call(
        _encode_stats_kernel,
        out_shape=(jax.ShapeDtypeStruct((rp, dout), jnp.float32),
                   jax.ShapeDtypeStruct((1, dout), jnp.float32),
                   jax.ShapeDtypeStruct((1, dout), jnp.float32)),
        grid_spec=pltpu.PrefetchScalarGridSpec(
            num_scalar_prefetch=0,
            grid=(tiles,),
            in_specs=[
                pl.BlockSpec((_TILE_ROWS, dout), lambda i: (i, 0)),
                pl.BlockSpec(wn.shape, lambda i: (0, 0)),
                pl.BlockSpec((1, dout), lambda i: (0, 0)),
                pl.BlockSpec((_TILE_ROWS, dout), lambda i: (i, 0)),
                pl.BlockSpec(we.shape, lambda i: (0, 0)),
            ],
            out_specs=[
                pl.BlockSpec((_TILE_ROWS, dout), lambda i: (i, 0)),
                pl.BlockSpec((1, dout), lambda i: (0, 0)),
                pl.BlockSpec((1, dout), lambda i: (0, 0)),
            ],
        ),
        compiler_params=pltpu.CompilerParams(
            dimension_semantics=("arbitrary",),
            vmem_limit_bytes=_VMEM_LIMIT_BYTES),
    )(xf, wn, bn, ef, we)
    x_enc = x_enc[:n] if rp != n else x_enc

    inv_cnt = 1.0 / float(max(r_e, 1))
    e_enc = pl.pallas_call(
        _make_edge_norm_kernel(inv_cnt, eps),
        out_shape=jax.ShapeDtypeStruct((rp, dout), jnp.float32),
        grid_spec=pltpu.PrefetchScalarGridSpec(
            num_scalar_prefetch=0,
            grid=(tiles,),
            in_specs=[
                pl.BlockSpec((_TILE_ROWS, dout), lambda i: (i, 0)),
                pl.BlockSpec(we.shape, lambda i: (0, 0)),
                pl.BlockSpec((1, dout), lambda i: (0, 0)),
                pl.BlockSpec((1, dout), lambda i: (0, 0)),
                pl.BlockSpec((1, dout), lambda i: (0, 0)),
                pl.BlockSpec((1, dout), lambda i: (0, 0)),
            ],
            out_specs=pl.BlockSpec((_TILE_ROWS, dout), lambda i: (i, 0)),
        ),
        compiler_params=pltpu.CompilerParams(
            dimension_semantics=("parallel",),
            vmem_limit_bytes=_VMEM_LIMIT_BYTES),
    )(ef, we, s1, s2, row(bn_gamma), row(bn_beta))
    e_enc = e_enc[:r_e] if rp != r_e else e_enc

    return {"x": x_enc, "edge_attr": e_enc}


# speedup vs baseline: 2.8406x; 1.0249x over previous
import jax
import jax.numpy as jnp
from jax import lax
from jax.experimental import pallas as pl
from jax.experimental.pallas import tpu as pltpu

_TILE_ROWS = 16384
_VMEM_LIMIT_BYTES = 56 * 1024 * 1024


def _cdiv(a, b):
    return -(-a // b)


def _pad_rows(a, rows_padded):
    r = a.shape[0]
    if rows_padded != r:
        a = jnp.pad(a, ((0, rows_padded - r), (0, 0)))
    return a


def _encode_stats_kernel(x_ref, wn_ref, bn_ref, e_ref, we_ref,
                         xo_ref, s1_ref, s2_ref):
    @pl.when(pl.program_id(0) == 0)
    def _init():
        s1_ref[...] = jnp.zeros_like(s1_ref)
        s2_ref[...] = jnp.zeros_like(s2_ref)

    xo_ref[...] = (jnp.dot(x_ref[...], wn_ref[...],
                           preferred_element_type=jnp.float32)
                   + bn_ref[...]).astype(xo_ref.dtype)
    acc = jnp.dot(e_ref[...], we_ref[...], preferred_element_type=jnp.float32)
    s1_ref[...] += jnp.sum(acc, axis=0, keepdims=True)
    s2_ref[...] += jnp.sum(acc * acc, axis=0, keepdims=True)


def _make_edge_norm_kernel(inv_cnt, eps):
    def _edge_norm_kernel(e_ref, we_ref, s1_ref, s2_ref, g_ref, b_ref, o_ref):
        mu = s1_ref[...] * inv_cnt
        var = jnp.maximum(s2_ref[...] * inv_cnt - mu * mu, 0.0)
        scale = g_ref[...] * lax.rsqrt(var + eps)
        shift = b_ref[...] - mu * scale
        acc = jnp.dot(e_ref[...], we_ref[...],
                      preferred_element_type=jnp.float32)
        o_ref[...] = (acc * scale + shift).astype(o_ref.dtype)
    return _edge_norm_kernel


def kernel(x, edge_attr, w_node, b_node, w_edge, b_edge, bn_gamma, bn_beta):
    eps = 1e-5
    n, _ = x.shape
    r_e, _ = edge_attr.shape
    dout = w_node.shape[1]

    tiles = max(_cdiv(max(n, r_e), _TILE_ROWS), 1)
    rp = tiles * _TILE_ROWS

    xf = _pad_rows(x.astype(jnp.float32), rp)
    ef = _pad_rows(edge_attr.astype(jnp.float32), rp)
    wn = w_node.astype(jnp.float32)
    we = w_edge.astype(jnp.float32)
    bn = b_node.astype(jnp.float32).reshape(1, dout)
    row = lambda v: v.astype(jnp.float32).reshape(1, dout)

    x_enc, s1, s2 = pl.pallas_call(
        _encode_stats_kernel,
        out_shape=(jax.ShapeDtypeStruct((rp, dout), jnp.float32),
                   jax.ShapeDtypeStruct((1, dout), jnp.float32),
                   jax.ShapeDtypeStruct((1, dout), jnp.float32)),
        grid_spec=pltpu.PrefetchScalarGridSpec(
            num_scalar_prefetch=0,
            grid=(tiles,),
            in_specs=[
                pl.BlockSpec((_TILE_ROWS, dout), lambda i: (i, 0)),
                pl.BlockSpec(wn.shape, lambda i: (0, 0)),
                pl.BlockSpec((1, dout), lambda i: (0, 0)),
                pl.BlockSpec((_TILE_ROWS, dout), lambda i: (i, 0)),
                pl.BlockSpec(we.shape, lambda i: (0, 0)),
            ],
            out_specs=[
                pl.BlockSpec((_TILE_ROWS, dout), lambda i: (i, 0)),
                pl.BlockSpec((1, dout), lambda i: (0, 0)),
                pl.BlockSpec((1, dout), lambda i: (0, 0)),
            ],
        ),
        compiler_params=pltpu.CompilerParams(
            dimension_semantics=("arbitrary",),
            vmem_limit_bytes=_VMEM_LIMIT_BYTES),
    )(xf, wn, bn, ef, we)
    x_enc = x_enc[:n] if rp != n else x_enc

    inv_cnt = 1.0 / float(max(r_e, 1))
    e_enc = pl.pallas_call(
        _make_edge_norm_kernel(inv_cnt, eps),
        out_shape=jax.ShapeDtypeStruct((rp, dout), jnp.float32),
        grid_spec=pltpu.PrefetchScalarGridSpec(
            num_scalar_prefetch=0,
            grid=(tiles,),
            in_specs=[
                pl.BlockSpec((_TILE_ROWS, dout), lambda i: (i, 0)),
                pl.BlockSpec(we.shape, lambda i: (0, 0)),
                pl.BlockSpec((1, dout), lambda i: (0, 0)),
                pl.BlockSpec((1, dout), lambda i: (0, 0)),
                pl.BlockSpec((1, dout), lambda i: (0, 0)),
                pl.BlockSpec((1, dout), lambda i: (0, 0)),
            ],
            out_specs=pl.BlockSpec((_TILE_ROWS, dout), lambda i: (i, 0)),
        ),
        compiler_params=pltpu.CompilerParams(
            dimension_semantics=("parallel",),
            vmem_limit_bytes=_VMEM_LIMIT_BYTES),
    )(ef, we, s1, s2, row(bn_gamma), row(bn_beta))
    e_enc = e_enc[:r_e] if rp != r_e else e_enc

    return {"x": x_enc, "edge_attr": e_enc}
